# baseline (speedup 1.0000x reference)
import jax
import jax.numpy as jnp
from jax import lax
from jax.experimental import pallas as pl
from jax.experimental.pallas import tpu as pltpu

_NBINS = 15
_NS = 4


def _body(b_ref, p0, p1, p2, p3, l0, l1, l2, l3, out_ref,
          cnt_ref, sc_ref, sa_ref):
    i = pl.program_id(0)
    nblk = pl.num_programs(0)

    @pl.when(i == 0)
    def _init():
        cnt_ref[...] = jnp.zeros_like(cnt_ref)
        sc_ref[...] = jnp.zeros_like(sc_ref)
        sa_ref[...] = jnp.zeros_like(sa_ref)

    b = b_ref[...]
    cnt_d = None
    sc_d = None
    sa_d = None
    for p_ref, l_ref in ((p0, l0), (p1, l1), (p2, l2), (p3, l3)):
        x = p_ref[0]
        bm = x.shape[0]
        xt = x.T
        conf = jnp.max(xt, axis=0, keepdims=True)

        cls_iota = lax.broadcasted_iota(jnp.int32, xt.shape, 0)
        first = jnp.min(jnp.where(xt == conf, cls_iota, 64),
                        axis=0, keepdims=True)
        accf = (first == l_ref[0, 0]).astype(jnp.float32)

        gt = conf > b
        c_d = jnp.where(gt, 1.0, 0.0)
        s_d = jnp.where(gt, conf, 0.0)
        a_d = jnp.where(gt, accf, 0.0)
        cnt_d = c_d if cnt_d is None else cnt_d + c_d
        sc_d = s_d if sc_d is None else sc_d + s_d
        sa_d = a_d if sa_d is None else sa_d + a_d

    cnt_ref[...] += cnt_d
    sc_ref[...] += sc_d
    sa_ref[...] += sa_d

    @pl.when(i == nblk - 1)
    def _finish():
        cum_c = jnp.sum(cnt_ref[...], axis=1, keepdims=True)
        cum_s = jnp.sum(sc_ref[...], axis=1, keepdims=True)
        cum_a = jnp.sum(sa_ref[...], axis=1, keepdims=True)
        cnt = cum_c[:_NBINS, :] - cum_c[1:, :]
        s_conf = cum_s[:_NBINS, :] - cum_s[1:, :]
        s_acc = cum_a[:_NBINS, :] - cum_a[1:, :]
        denom = jnp.maximum(cnt, 1.0)
        ce = jnp.abs(s_conf / denom - s_acc / denom)
        ce = jnp.where(cnt > 0.0, ce, 0.0)
        out_ref[...] = jnp.max(ce, axis=(0, 1), keepdims=True)


def kernel(softmaxes_probs, labels):
    n, c = softmaxes_probs.shape
    rows = n // _NS
    bm = next(b for b in (10000, 5000, 1000, rows) if rows % b == 0)
    nblk = rows // bm

    bounds = jnp.linspace(0.0, 1.0, _NBINS + 1).reshape(_NBINS + 1, 1)
    pv = softmaxes_probs.reshape(_NS, rows, c)
    lv = labels.astype(jnp.int32).reshape(_NS, nblk, 1, bm)

    def pspec(s):
        return pl.BlockSpec((1, bm, c), lambda i, s=s: (s, i, 0))

    def lspec(s):
        return pl.BlockSpec((1, 1, 1, bm), lambda i, s=s: (s, i, 0, 0))

    out = pl.pallas_call(
        _body,
        grid=(nblk,),
        in_specs=(
            [pl.BlockSpec((_NBINS + 1, 1), lambda i: (0, 0))]
            + [pspec(s) for s in range(_NS)]
            + [lspec(s) for s in range(_NS)]
        ),
        out_specs=pl.BlockSpec((1, 1), lambda i: (0, 0)),
        out_shape=jax.ShapeDtypeStruct((1, 1), jnp.float32),
        scratch_shapes=[
            pltpu.VMEM((_NBINS + 1, bm), jnp.float32),
            pltpu.VMEM((_NBINS + 1, bm), jnp.float32),
            pltpu.VMEM((_NBINS + 1, bm), jnp.float32),
        ],
        compiler_params=pltpu.CompilerParams(
            dimension_semantics=("arbitrary",),
        ),
    )(bounds, *([pv] * _NS), *([lv] * _NS))
    return out.reshape(1)
